# baseline (speedup 1.0000x reference)
import functools

import jax
import jax.numpy as jnp
from jax.experimental import pallas as pl
from jax.experimental.pallas import tpu as pltpu

N = 10000
E = 320000
D = 128
DE = 16
G = 128

_EB = 4000
_NB = 1000


def _emb_body(ea_ref, w_ref, b_ref, out_ref):
    out_ref[...] = (
        jnp.dot(ea_ref[...], w_ref[...], preferred_element_type=jnp.float32)
        + b_ref[...]
    )


def _edge_emb(edge_attr, W_edge, b_edge):
    return pl.pallas_call(
        _emb_body,
        grid=(E // _EB,),
        in_specs=[
            pl.BlockSpec((_EB, DE), lambda i: (i, 0)),
            pl.BlockSpec((DE, D), lambda i: (0, 0)),
            pl.BlockSpec((1, D), lambda i: (0, 0)),
        ],
        out_specs=pl.BlockSpec((_EB, D), lambda i: (i, 0)),
        out_shape=jax.ShapeDtypeStruct((E, D), jnp.float32),
    )(edge_attr, W_edge, b_edge.reshape(1, D))


def _tail_body(agg_ref, batch_ref, wg_ref, bg_ref, w1_ref, b1_ref,
               w2_ref, b2_ref, out_ref, sums_ref, cnt_ref):
    i = pl.program_id(0)
    nb = pl.num_programs(0)

    @pl.when(i == 0)
    def _():
        sums_ref[...] = jnp.zeros_like(sums_ref)
        cnt_ref[...] = jnp.zeros_like(cnt_ref)

    h = jnp.maximum(
        jnp.dot(agg_ref[...], wg_ref[...], preferred_element_type=jnp.float32)
        + bg_ref[...], 0.0)
    b = batch_ref[0, 0, :]
    gi = jax.lax.broadcasted_iota(jnp.int32, (_NB, G), 1)
    onehot = jnp.where(b[:, None] == gi, 1.0, 0.0)
    sums_ref[...] += jax.lax.dot_general(
        onehot, h, (((0,), (0,)), ((), ())), preferred_element_type=jnp.float32)
    cnt_ref[...] += jax.lax.dot_general(
        onehot, jnp.ones((_NB, D), jnp.float32), (((0,), (0,)), ((), ())),
        preferred_element_type=jnp.float32)

    @pl.when(i == nb - 1)
    def _():
        pooled = sums_ref[...] / jnp.maximum(cnt_ref[...], 1.0)
        t = jnp.maximum(
            jnp.dot(pooled, w1_ref[...], preferred_element_type=jnp.float32)
            + b1_ref[...], 0.0)
        out_ref[...] = (
            jnp.dot(t, w2_ref[...], preferred_element_type=jnp.float32)
            + b2_ref[...])


def _tail(agg, batch32, W_gnn, b_gnn, W1, b1, W2, b2):
    nblocks = N // _NB
    return pl.pallas_call(
        _tail_body,
        grid=(nblocks,),
        in_specs=[
            pl.BlockSpec((_NB, D), lambda i: (i, 0)),
            pl.BlockSpec((1, 1, _NB), lambda i: (i, 0, 0)),
            pl.BlockSpec((D, D), lambda i: (0, 0)),
            pl.BlockSpec((1, D), lambda i: (0, 0)),
            pl.BlockSpec((D, D), lambda i: (0, 0)),
            pl.BlockSpec((1, D), lambda i: (0, 0)),
            pl.BlockSpec((D, D), lambda i: (0, 0)),
            pl.BlockSpec((1, D), lambda i: (0, 0)),
        ],
        out_specs=pl.BlockSpec((G, D), lambda i: (0, 0)),
        out_shape=jax.ShapeDtypeStruct((G, D), jnp.float32),
        scratch_shapes=[
            pltpu.VMEM((G, D), jnp.float32),
            pltpu.VMEM((G, D), jnp.float32),
        ],
    )(agg, batch32.reshape(nblocks, 1, _NB), W_gnn, b_gnn.reshape(1, D),
      W1, b1.reshape(1, D), W2, b2.reshape(1, D))


def kernel(x, edge_index, edge_attr, batch, W_edge, b_edge, W_gnn, b_gnn,
           W1, b1, W2, b2):
    src = edge_index[0].astype(jnp.int32)
    dst = edge_index[1].astype(jnp.int32)
    batch32 = batch.astype(jnp.int32)

    emb = _edge_emb(edge_attr, W_edge, b_edge)

    msg = jax.nn.relu(x[src] + emb)
    agg = jax.ops.segment_sum(msg, dst, num_segments=N)

    return _tail(agg, batch32, W_gnn, b_gnn, W1, b1, W2, b2)


# speedup vs baseline: 2.7547x; 2.7547x over previous
import functools

import jax
import jax.numpy as jnp
from jax import lax
from jax.experimental import pallas as pl
from jax.experimental.pallas import tpu as pltpu
from jax.experimental.pallas import tpu_sc as plsc

N = 10000
E = 320000
D = 128
DE = 16
G = 128

_EB = 4000
_NB = 1000

_NC = 2
_NS = 16
_NW = _NC * _NS
_C = 80
_EPW = E // _NW
_CHUNKS = _EPW // _C
_NP = 10240
_RPS = _NP // _NS
_ZR = 128


def _emb_body(ea_ref, w_ref, b_ref, out_ref):
    out_ref[...] = (
        jnp.dot(ea_ref[...], w_ref[...], preferred_element_type=jnp.float32)
        + b_ref[...]
    )


def _edge_emb(edge_attr, W_edge, b_edge):
    return pl.pallas_call(
        _emb_body,
        grid=(E // _EB,),
        in_specs=[
            pl.BlockSpec((_EB, DE), lambda i: (i, 0)),
            pl.BlockSpec((DE, D), lambda i: (0, 0)),
            pl.BlockSpec((1, D), lambda i: (0, 0)),
        ],
        out_specs=pl.BlockSpec((_EB, D), lambda i: (i, 0)),
        out_shape=jax.ShapeDtypeStruct((E, D), jnp.float32),
    )(edge_attr, W_edge, b_edge.reshape(1, D))


def _sc_mid_body(x_hbm, src_hbm, dst_hbm, emb_hbm, out_hbm,
                 idx_s, idx_d, xrows, embv, zbuf, agg, sem):
    c = lax.axis_index("c")
    s = lax.axis_index("s")
    wid = s * _NC + c

    zero16 = jnp.zeros((16,), jnp.float32)

    def zrow(r, carry):
        for k in range(8):
            zbuf[r, pl.ds(k * 16, 16)] = zero16
        return carry
    lax.fori_loop(0, _ZR, zrow, 0)

    def zcp(k, carry):
        pltpu.sync_copy(zbuf, agg.at[pl.ds(s * _RPS + k * _ZR, _ZR)])
        return carry
    lax.fori_loop(0, _RPS // _ZR, zcp, 0)
    plsc.subcore_barrier()

    base0 = wid * _EPW

    def chunk(j, carry):
        base = base0 + j * _C
        pltpu.sync_copy(src_hbm.at[pl.ds(base, _C)], idx_s)
        pltpu.sync_copy(dst_hbm.at[pl.ds(base, _C)], idx_d)
        pltpu.async_copy(x_hbm.at[idx_s], xrows, sem).wait()
        pltpu.sync_copy(emb_hbm.at[pl.ds(base, _C)], embv)

        def row(r, carry2):
            for k in range(8):
                sl = pl.ds(k * 16, 16)
                xrows[r, sl] = jnp.maximum(xrows[r, sl] + embv[r, sl], 0.0)
            return carry2
        lax.fori_loop(0, _C, row, 0)
        pltpu.sync_copy(xrows, agg.at[idx_d], add=True)
        return carry
    lax.fori_loop(0, _CHUNKS, chunk, 0)
    plsc.subcore_barrier()

    def drain(k, carry):
        off = s * _RPS + k * _ZR
        pltpu.sync_copy(agg.at[pl.ds(off, _ZR)], zbuf)
        pltpu.sync_copy(zbuf, out_hbm.at[c, pl.ds(off, _ZR)])
        return carry
    lax.fori_loop(0, _RPS // _ZR, drain, 0)


def _sc_mid(x, src, dst, emb):
    f = functools.partial(
        pl.kernel,
        mesh=plsc.VectorSubcoreMesh(core_axis_name="c", subcore_axis_name="s"),
        out_type=jax.ShapeDtypeStruct((_NC, _NP, D), jnp.float32),
        scratch_types=[
            pltpu.VMEM((_C,), jnp.int32),
            pltpu.VMEM((_C,), jnp.int32),
            pltpu.VMEM((_C, D), jnp.float32),
            pltpu.VMEM((_C, D), jnp.float32),
            pltpu.VMEM((_ZR, D), jnp.float32),
            pltpu.VMEM_SHARED((_NP, D), jnp.float32),
            pltpu.SemaphoreType.DMA,
        ],
    )(_sc_mid_body)
    return f(x, src, dst, emb)


def _tail_body(agg_ref, batch_ref, wg_ref, bg_ref, w1_ref, b1_ref,
               w2_ref, b2_ref, out_ref, sums_ref, cnt_ref):
    i = pl.program_id(0)
    nb = pl.num_programs(0)

    @pl.when(i == 0)
    def _():
        sums_ref[...] = jnp.zeros_like(sums_ref)
        cnt_ref[...] = jnp.zeros_like(cnt_ref)

    agg = agg_ref[0] + agg_ref[1]
    h = jnp.maximum(
        jnp.dot(agg, wg_ref[...], preferred_element_type=jnp.float32)
        + bg_ref[...], 0.0)
    b = batch_ref[0, 0, :]
    gi = jax.lax.broadcasted_iota(jnp.int32, (_NB, G), 1)
    onehot = jnp.where(b[:, None] == gi, 1.0, 0.0)
    sums_ref[...] += jax.lax.dot_general(
        onehot, h, (((0,), (0,)), ((), ())), preferred_element_type=jnp.float32)
    cnt_ref[...] += jax.lax.dot_general(
        onehot, jnp.ones((_NB, D), jnp.float32), (((0,), (0,)), ((), ())),
        preferred_element_type=jnp.float32)

    @pl.when(i == nb - 1)
    def _():
        pooled = sums_ref[...] / jnp.maximum(cnt_ref[...], 1.0)
        t = jnp.maximum(
            jnp.dot(pooled, w1_ref[...], preferred_element_type=jnp.float32)
            + b1_ref[...], 0.0)
        out_ref[...] = (
            jnp.dot(t, w2_ref[...], preferred_element_type=jnp.float32)
            + b2_ref[...])


def _tail(agg, batch32, W_gnn, b_gnn, W1, b1, W2, b2):
    nblocks = N // _NB
    return pl.pallas_call(
        _tail_body,
        grid=(nblocks,),
        in_specs=[
            pl.BlockSpec((_NC, _NB, D), lambda i: (0, i, 0)),
            pl.BlockSpec((1, 1, _NB), lambda i: (i, 0, 0)),
            pl.BlockSpec((D, D), lambda i: (0, 0)),
            pl.BlockSpec((1, D), lambda i: (0, 0)),
            pl.BlockSpec((D, D), lambda i: (0, 0)),
            pl.BlockSpec((1, D), lambda i: (0, 0)),
            pl.BlockSpec((D, D), lambda i: (0, 0)),
            pl.BlockSpec((1, D), lambda i: (0, 0)),
        ],
        out_specs=pl.BlockSpec((G, D), lambda i: (0, 0)),
        out_shape=jax.ShapeDtypeStruct((G, D), jnp.float32),
        scratch_shapes=[
            pltpu.VMEM((G, D), jnp.float32),
            pltpu.VMEM((G, D), jnp.float32),
        ],
    )(agg, batch32.reshape(nblocks, 1, _NB), W_gnn, b_gnn.reshape(1, D),
      W1, b1.reshape(1, D), W2, b2.reshape(1, D))


def kernel(x, edge_index, edge_attr, batch, W_edge, b_edge, W_gnn, b_gnn,
           W1, b1, W2, b2):
    src = edge_index[0].astype(jnp.int32)
    dst = edge_index[1].astype(jnp.int32)
    batch32 = batch.astype(jnp.int32)

    emb = _edge_emb(edge_attr, W_edge, b_edge)
    agg2 = _sc_mid(x, src, dst, emb)
    return _tail(agg2, batch32, W_gnn, b_gnn, W1, b1, W2, b2)


# speedup vs baseline: 4.1095x; 1.4918x over previous
import functools

import jax
import jax.numpy as jnp
from jax import lax
from jax.experimental import pallas as pl
from jax.experimental.pallas import tpu as pltpu
from jax.experimental.pallas import tpu_sc as plsc

N = 10000
E = 320000
D = 128
DE = 16
G = 128

_EB = 4000
_NB = 1000

_NC = 2
_NS = 16
_NW = _NC * _NS
_C = 80
_EPW = E // _NW
_CHUNKS = _EPW // _C
_NP = 10240
_RPS = _NP // _NS
_ZR = _C


def _emb_body(ea_ref, w_ref, b_ref, out_ref):
    out_ref[...] = (
        jnp.dot(ea_ref[...], w_ref[...], preferred_element_type=jnp.float32)
        + b_ref[...]
    )


def _edge_emb(edge_attr, W_edge, b_edge):
    return pl.pallas_call(
        _emb_body,
        grid=(E // _EB,),
        in_specs=[
            pl.BlockSpec((_EB, DE), lambda i: (i, 0)),
            pl.BlockSpec((DE, D), lambda i: (0, 0)),
            pl.BlockSpec((1, D), lambda i: (0, 0)),
        ],
        out_specs=pl.BlockSpec((_EB, D), lambda i: (i, 0)),
        out_shape=jax.ShapeDtypeStruct((E, D), jnp.float32),
    )(edge_attr, W_edge, b_edge.reshape(1, D))


def _sc_mid_body(x_hbm, src_hbm, dst_hbm, emb_hbm, out_hbm,
                 idx_sA, idx_dA, idx_sB, idx_dB, xrA, xrB, embA, embB,
                 agg, gsemA, esemA, gsemB, esemB):
    c = lax.axis_index("c")
    s = lax.axis_index("s")
    wid = s * _NC + c
    base0 = wid * _EPW

    def prefetch(j, idx_s, idx_d, xr, emb, gsem, esem):
        base = base0 + j * _C
        pltpu.sync_copy(src_hbm.at[pl.ds(base, _C)], idx_s)
        pltpu.sync_copy(dst_hbm.at[pl.ds(base, _C)], idx_d)
        pltpu.async_copy(x_hbm.at[idx_s], xr, gsem)
        pltpu.async_copy(emb_hbm.at[pl.ds(base, _C)], emb, esem)

    def consume(j, idx_s, idx_d, xr, emb, gsem, esem):
        base = base0 + j * _C
        pltpu.make_async_copy(x_hbm.at[idx_s], xr, gsem).wait()
        pltpu.make_async_copy(emb_hbm.at[pl.ds(base, _C)], emb, esem).wait()

        def row(r, carry2):
            for k in range(8):
                sl = pl.ds(k * 16, 16)
                xr[r, sl] = jnp.maximum(xr[r, sl] + emb[r, sl], 0.0)
            return carry2
        lax.fori_loop(0, _C, row, 0)
        pltpu.sync_copy(xr, agg.at[idx_d], add=True)

    A = (idx_sA, idx_dA, xrA, embA, gsemA, esemA)
    B = (idx_sB, idx_dB, xrB, embB, gsemB, esemB)

    prefetch(0, *A)

    zero16 = jnp.zeros((16,), jnp.float32)

    def zrow(r, carry):
        for k in range(8):
            xrB[r, pl.ds(k * 16, 16)] = zero16
        return carry
    lax.fori_loop(0, _ZR, zrow, 0)

    def zcp(k, carry):
        pltpu.sync_copy(xrB, agg.at[pl.ds(s * _RPS + k * _ZR, _ZR)])
        return carry
    lax.fori_loop(0, _RPS // _ZR, zcp, 0)
    plsc.subcore_barrier()

    def pair(t, carry):
        j0 = 2 * t
        j1 = j0 + 1

        @pl.when(j1 < _CHUNKS)
        def _():
            prefetch(j1, *B)
        consume(j0, *A)

        @pl.when(j0 + 2 < _CHUNKS)
        def _():
            prefetch(j0 + 2, *A)

        @pl.when(j1 < _CHUNKS)
        def _():
            consume(j1, *B)
        return carry
    lax.fori_loop(0, (_CHUNKS + 1) // 2, pair, 0)
    plsc.subcore_barrier()

    def drain(k, carry):
        off = s * _RPS + k * _ZR
        pltpu.sync_copy(agg.at[pl.ds(off, _ZR)], xrA)
        pltpu.sync_copy(xrA, out_hbm.at[c, pl.ds(off, _ZR)])
        return carry
    lax.fori_loop(0, _RPS // _ZR, drain, 0)


def _sc_mid(x, src, dst, emb):
    f = functools.partial(
        pl.kernel,
        mesh=plsc.VectorSubcoreMesh(core_axis_name="c", subcore_axis_name="s"),
        out_type=jax.ShapeDtypeStruct((_NC, _NP, D), jnp.float32),
        scratch_types=[
            pltpu.VMEM((_C,), jnp.int32),
            pltpu.VMEM((_C,), jnp.int32),
            pltpu.VMEM((_C,), jnp.int32),
            pltpu.VMEM((_C,), jnp.int32),
            pltpu.VMEM((_C, D), jnp.float32),
            pltpu.VMEM((_C, D), jnp.float32),
            pltpu.VMEM((_C, D), jnp.float32),
            pltpu.VMEM((_C, D), jnp.float32),
            pltpu.VMEM_SHARED((_NP, D), jnp.float32),
            pltpu.SemaphoreType.DMA,
            pltpu.SemaphoreType.DMA,
            pltpu.SemaphoreType.DMA,
            pltpu.SemaphoreType.DMA,
        ],
    )(_sc_mid_body)
    return f(x, src, dst, emb)


def _tail_body(agg_ref, batch_ref, wg_ref, bg_ref, w1_ref, b1_ref,
               w2_ref, b2_ref, out_ref, sums_ref, cnt_ref):
    i = pl.program_id(0)
    nb = pl.num_programs(0)

    @pl.when(i == 0)
    def _():
        sums_ref[...] = jnp.zeros_like(sums_ref)
        cnt_ref[...] = jnp.zeros_like(cnt_ref)

    agg = agg_ref[0] + agg_ref[1]
    h = jnp.maximum(
        jnp.dot(agg, wg_ref[...], preferred_element_type=jnp.float32)
        + bg_ref[...], 0.0)
    b = batch_ref[0, 0, :]
    gi = jax.lax.broadcasted_iota(jnp.int32, (_NB, G), 1)
    onehot = jnp.where(b[:, None] == gi, 1.0, 0.0)
    sums_ref[...] += jax.lax.dot_general(
        onehot, h, (((0,), (0,)), ((), ())), preferred_element_type=jnp.float32)
    cnt_ref[...] += jax.lax.dot_general(
        onehot, jnp.ones((_NB, D), jnp.float32), (((0,), (0,)), ((), ())),
        preferred_element_type=jnp.float32)

    @pl.when(i == nb - 1)
    def _():
        pooled = sums_ref[...] / jnp.maximum(cnt_ref[...], 1.0)
        t = jnp.maximum(
            jnp.dot(pooled, w1_ref[...], preferred_element_type=jnp.float32)
            + b1_ref[...], 0.0)
        out_ref[...] = (
            jnp.dot(t, w2_ref[...], preferred_element_type=jnp.float32)
            + b2_ref[...])


def _tail(agg, batch32, W_gnn, b_gnn, W1, b1, W2, b2):
    nblocks = N // _NB
    return pl.pallas_call(
        _tail_body,
        grid=(nblocks,),
        in_specs=[
            pl.BlockSpec((_NC, _NB, D), lambda i: (0, i, 0)),
            pl.BlockSpec((1, 1, _NB), lambda i: (i, 0, 0)),
            pl.BlockSpec((D, D), lambda i: (0, 0)),
            pl.BlockSpec((1, D), lambda i: (0, 0)),
            pl.BlockSpec((D, D), lambda i: (0, 0)),
            pl.BlockSpec((1, D), lambda i: (0, 0)),
            pl.BlockSpec((D, D), lambda i: (0, 0)),
            pl.BlockSpec((1, D), lambda i: (0, 0)),
        ],
        out_specs=pl.BlockSpec((G, D), lambda i: (0, 0)),
        out_shape=jax.ShapeDtypeStruct((G, D), jnp.float32),
        scratch_shapes=[
            pltpu.VMEM((G, D), jnp.float32),
            pltpu.VMEM((G, D), jnp.float32),
        ],
    )(agg, batch32.reshape(nblocks, 1, _NB), W_gnn, b_gnn.reshape(1, D),
      W1, b1.reshape(1, D), W2, b2.reshape(1, D))


def kernel(x, edge_index, edge_attr, batch, W_edge, b_edge, W_gnn, b_gnn,
           W1, b1, W2, b2):
    src = edge_index[0].astype(jnp.int32)
    dst = edge_index[1].astype(jnp.int32)
    batch32 = batch.astype(jnp.int32)

    emb = _edge_emb(edge_attr, W_edge, b_edge)
    agg2 = _sc_mid(x, src, dst, emb)
    return _tail(agg2, batch32, W_gnn, b_gnn, W1, b1, W2, b2)
